# baseline (speedup 1.0000x reference)
import functools
import math

import jax
import jax.numpy as jnp
from jax import lax
from jax.experimental import pallas as pl
from jax.experimental.pallas import tpu as pltpu
from jax.experimental.pallas import tpu_sc as plsc

EMB = 64
SCALE = 8.0
LANES = 16

NC = 2
NS = 16
NW = NC * NS

NBATCH = 4096
SEQ = 200
B_TOTAL = NBATCH * SEQ
ROWS_PER_W = B_TOTAL // NW
BATCH_PER_W = NBATCH // NW
CHUNK = 128
NBUF = 4

_mesh = plsc.VectorSubcoreMesh(
    core_axis_name="c", subcore_axis_name="s", num_cores=NC, num_subcores=NS)


@functools.partial(
    pl.kernel,
    out_type=jax.ShapeDtypeStruct((NBATCH, SEQ, EMB), jnp.float32),
    mesh=_mesh,
    scratch_types=[
        pltpu.VMEM((ROWS_PER_W,), jnp.int32),
    ] + [pltpu.VMEM((SEQ, EMB), jnp.float32) for _ in range(NBUF)]
      + [pltpu.SemaphoreType.DMA for _ in range(2 * NBUF)],
    compiler_params=pltpu.CompilerParams(use_tc_tiling_on_sc=False),
)
def _emb_lookup(tok_hbm, table_hbm, out_hbm, idx_v,
                a0, a1, a2, a3,
                g0, g1, g2, g3, o0, o1, o2, o3):
    bufs = [a0, a1, a2, a3]
    gsems = [g0, g1, g2, g3]
    osems = [o0, o1, o2, o3]

    wid = lax.axis_index("s") * NC + lax.axis_index("c")
    pltpu.sync_copy(tok_hbm.at[pl.ds(wid * ROWS_PER_W, ROWS_PER_W)], idx_v)
    batch_base = wid * BATCH_PER_W

    def fire_gather(k, j):
        pltpu.async_copy(
            table_hbm.at[idx_v.at[pl.ds(k * SEQ, CHUNK)]],
            bufs[j].at[pl.ds(0, CHUNK)], gsems[j])
        pltpu.async_copy(
            table_hbm.at[idx_v.at[pl.ds(k * SEQ + CHUNK, SEQ - CHUNK)]],
            bufs[j].at[pl.ds(CHUNK, SEQ - CHUNK)], gsems[j])

    def wait_gather(j):
        pltpu.make_async_copy(
            table_hbm.at[idx_v.at[pl.ds(0, CHUNK)]],
            bufs[j].at[pl.ds(0, CHUNK)], gsems[j]).wait()
        pltpu.make_async_copy(
            table_hbm.at[idx_v.at[pl.ds(0, SEQ - CHUNK)]],
            bufs[j].at[pl.ds(CHUNK, SEQ - CHUNK)], gsems[j]).wait()

    def fire_out(k, j):
        pltpu.async_copy(bufs[j], out_hbm.at[batch_base + k], osems[j])

    def wait_out(j):
        pltpu.make_async_copy(bufs[j], out_hbm.at[0], osems[j]).wait()

    def scale(j):
        buf = bufs[j]

        def body(r, carry):
            for l in range(EMB // LANES):
                sl = pl.ds(l * LANES, LANES)
                buf[r, sl] = buf[r, sl] * SCALE
            return carry

        lax.fori_loop(0, SEQ, body, 0, unroll=8)

    fire_gather(0, 0)
    fire_gather(1, 1)

    for k in (0, 1):
        fire_gather(k + 2, (k + 2) % NBUF)
        wait_gather(k % NBUF)
        scale(k % NBUF)
        fire_out(k, k % NBUF)

    def loop_body(t, carry):
        for jj in range(NBUF):
            k = NBUF * t + 2 + jj
            j = (2 + jj) % NBUF
            wait_out(jj)
            fire_gather(k + 2, jj)
            wait_gather(j)
            scale(j)
            fire_out(k, j)
        return carry

    lax.fori_loop(0, (BATCH_PER_W - NBUF) // NBUF, loop_body, 0)

    for k in (BATCH_PER_W - 2, BATCH_PER_W - 1):
        wait_out((k + 2) % NBUF)
        wait_gather(k % NBUF)
        scale(k % NBUF)
        fire_out(k, k % NBUF)

    wait_out((BATCH_PER_W - 2) % NBUF)
    wait_out((BATCH_PER_W - 1) % NBUF)


def kernel(tokens, embedding):
    tok = jnp.maximum(tokens.astype(jnp.int32), 0).reshape(B_TOTAL)
    return _emb_lookup(tok, embedding)


# speedup vs baseline: 1.0001x; 1.0001x over previous
import functools
import math

import jax
import jax.numpy as jnp
from jax import lax
from jax.experimental import pallas as pl
from jax.experimental.pallas import tpu as pltpu
from jax.experimental.pallas import tpu_sc as plsc

EMB = 64
SCALE = 8.0
LANES = 16

NC = 2
NS = 16
NW = NC * NS

NBATCH = 4096
SEQ = 200
B_TOTAL = NBATCH * SEQ
ROWS_PER_W = B_TOTAL // NW
BATCH_PER_W = NBATCH // NW
CHUNK = 128
NBUF = 4

_mesh = plsc.VectorSubcoreMesh(
    core_axis_name="c", subcore_axis_name="s", num_cores=NC, num_subcores=NS)


@functools.partial(
    pl.kernel,
    out_type=jax.ShapeDtypeStruct((NBATCH, SEQ, EMB), jnp.float32),
    mesh=_mesh,
    scratch_types=[
        pltpu.VMEM((ROWS_PER_W,), jnp.int32),
    ] + [pltpu.VMEM((SEQ, EMB), jnp.float32) for _ in range(NBUF)]
      + [pltpu.SemaphoreType.DMA for _ in range(2 * NBUF)],
    compiler_params=pltpu.CompilerParams(
        use_tc_tiling_on_sc=False, needs_layout_passes=True),
)
def _emb_lookup(tok_hbm, table_hbm, out_hbm, idx_v,
                a0, a1, a2, a3,
                g0, g1, g2, g3, o0, o1, o2, o3):
    bufs = [a0, a1, a2, a3]
    gsems = [g0, g1, g2, g3]
    osems = [o0, o1, o2, o3]

    wid = lax.axis_index("s") * NC + lax.axis_index("c")
    pltpu.sync_copy(tok_hbm.at[pl.ds(wid * ROWS_PER_W, ROWS_PER_W)], idx_v)
    batch_base = wid * BATCH_PER_W

    def fire_gather(k, j):
        pltpu.async_copy(
            table_hbm.at[idx_v.at[pl.ds(k * SEQ, CHUNK)]],
            bufs[j].at[pl.ds(0, CHUNK)], gsems[j])
        pltpu.async_copy(
            table_hbm.at[idx_v.at[pl.ds(k * SEQ + CHUNK, SEQ - CHUNK)]],
            bufs[j].at[pl.ds(CHUNK, SEQ - CHUNK)], gsems[j])

    def wait_gather(j):
        pltpu.make_async_copy(
            table_hbm.at[idx_v.at[pl.ds(0, CHUNK)]],
            bufs[j].at[pl.ds(0, CHUNK)], gsems[j]).wait()
        pltpu.make_async_copy(
            table_hbm.at[idx_v.at[pl.ds(0, SEQ - CHUNK)]],
            bufs[j].at[pl.ds(CHUNK, SEQ - CHUNK)], gsems[j]).wait()

    def fire_out(k, j):
        pltpu.async_copy(bufs[j], out_hbm.at[batch_base + k], osems[j])

    def wait_out(j):
        pltpu.make_async_copy(bufs[j], out_hbm.at[0], osems[j]).wait()

    def scale(j):
        buf = bufs[j]

        def body(r, carry):
            for l in range(EMB // LANES):
                sl = pl.ds(l * LANES, LANES)
                buf[r, sl] = buf[r, sl] * SCALE
            return carry

        lax.fori_loop(0, SEQ, body, 0, unroll=8)

    fire_gather(0, 0)
    fire_gather(1, 1)

    for k in (0, 1):
        fire_gather(k + 2, (k + 2) % NBUF)
        wait_gather(k % NBUF)
        scale(k % NBUF)
        fire_out(k, k % NBUF)

    def loop_body(t, carry):
        for jj in range(NBUF):
            k = NBUF * t + 2 + jj
            j = (2 + jj) % NBUF
            wait_out(jj)
            fire_gather(k + 2, jj)
            wait_gather(j)
            scale(j)
            fire_out(k, j)
        return carry

    lax.fori_loop(0, (BATCH_PER_W - NBUF) // NBUF, loop_body, 0)

    for k in (BATCH_PER_W - 2, BATCH_PER_W - 1):
        wait_out((k + 2) % NBUF)
        wait_gather(k % NBUF)
        scale(k % NBUF)
        fire_out(k, k % NBUF)

    wait_out((BATCH_PER_W - 2) % NBUF)
    wait_out((BATCH_PER_W - 1) % NBUF)


def kernel(tokens, embedding):
    tok = jnp.maximum(tokens.astype(jnp.int32), 0).reshape(B_TOTAL)
    return _emb_lookup(tok, embedding)
